# baseline (speedup 1.0000x reference)
import jax
import jax.numpy as jnp
from jax import lax
from jax.experimental import pallas as pl
from jax.experimental.pallas import tpu as pltpu
from jax.experimental.pallas import tpu_sc as plsc

B, T, C = 16, 2048, 1024
TOUT = T + 2
PADDING_IDX = 1
PRE_SCALE = 7.0
EPS = 1e-5
L = 16
NJ = C // L
NW = 32
R = 32
NQ = T // (NW * R)


def _f(x):
    return jnp.float32(x)


def _rsqrt_vec(a_scalar):
    av = jnp.full((L,), a_scalar, jnp.float32)
    ii = lax.bitcast_convert_type(av, jnp.int32)
    y = lax.bitcast_convert_type(jnp.int32(0x5F3759DF) - (ii >> 1), jnp.float32)
    for _ in range(3):
        y = y * (_f(1.5) - _f(0.5) * av * y * y)
    return y


def _row_normalize(buf, r, g_buf, b_buf):
    def stats(j, carry):
        s, ss = carry
        v = buf[r, pl.ds(j * L, L)]
        return s + v, ss + v * v
    z = jnp.zeros((L,), jnp.float32)
    s, ss = lax.fori_loop(0, NJ, stats, (z, z))
    mean = jnp.sum(s) * _f(1.0 / C)
    var = jnp.sum(ss) * _f(1.0 / C) - mean * mean
    inv = _rsqrt_vec(var + _f(EPS))
    mv = jnp.full((L,), mean, jnp.float32)

    def norm(j, _):
        sl = pl.ds(j * L, L)
        buf[r, sl] = (buf[r, sl] - mv) * inv * g_buf[0, sl] + b_buf[0, sl]
        return 0
    lax.fori_loop(0, NJ, norm, 0)


def _fill_row(dst, rd, fn):
    def body(j, _):
        sl = pl.ds(j * L, L)
        dst[rd, sl] = fn(sl)
        return 0
    lax.fori_loop(0, NJ, body, 0)


def _sc_body(x_hbm, len_hbm, posm_hbm, pad_hbm, bos_hbm, eos_hbm,
             gsp_hbm, bsp_hbm, g_hbm, b_hbm, out_hbm,
             x_buf, pos_buf, out_buf, gv, bv, gsv, bsv,
             bos_n, eos_n, pad_row, lenv):
    wid = lax.axis_index("s") * 2 + lax.axis_index("c")

    pltpu.sync_copy(g_hbm, gv)
    pltpu.sync_copy(b_hbm, bv)
    pltpu.sync_copy(gsp_hbm, gsv)
    pltpu.sync_copy(bsp_hbm, bsv)
    pltpu.sync_copy(bos_hbm, bos_n)
    pltpu.sync_copy(eos_hbm, eos_n)
    pltpu.sync_copy(pad_hbm, pad_row)
    pltpu.sync_copy(len_hbm, lenv)

    _row_normalize(bos_n, 0, gsv, bsv)
    _row_normalize(eos_n, 0, gsv, bsv)

    def get_len(b):
        lbv = plsc.load_gather(lenv, [jnp.full((L,), b, jnp.int32)])
        return jnp.max(lbv)

    def process_row(r, t, lb, xrow):
        _fill_row(out_buf, r,
                  lambda sl: x_buf[xrow, sl] * _f(PRE_SCALE) + pos_buf[r, sl])

        @pl.when(t == 0)
        def _():
            _fill_row(out_buf, r,
                      lambda sl: bos_n[0, sl] * _f(PRE_SCALE) + pos_buf[r, sl])

        @pl.when(t == lb + 1)
        def _():
            _fill_row(out_buf, r,
                      lambda sl: eos_n[0, sl] * _f(PRE_SCALE) + pos_buf[r, sl])

        @pl.when(t > lb + 1)
        def _():
            _fill_row(out_buf, r,
                      lambda sl: x_buf[xrow, sl] * _f(PRE_SCALE) + pad_row[0, sl])

        _row_normalize(out_buf, r, gv, bv)

    for q in range(NQ):
        o0 = wid * (NQ * R) + q * R
        pltpu.sync_copy(posm_hbm.at[pl.ds(o0, R), :], pos_buf)

        def per_batch(b, _):
            lb = get_len(b)

            @pl.when(o0 == 0)
            def _():
                pltpu.sync_copy(x_hbm.at[b, pl.ds(0, R), :],
                                x_buf.at[pl.ds(8, R)])

            @pl.when(o0 > 0)
            def _():
                pltpu.sync_copy(x_hbm.at[b, pl.ds(o0 - 8, R + 8), :], x_buf)

            def per_row(r, _):
                process_row(r, o0 + r, lb, r + 7)
                return 0
            lax.fori_loop(0, R, per_row, 0)

            pltpu.sync_copy(out_buf, out_hbm.at[b, pl.ds(o0, R), :])
            return 0

        lax.fori_loop(0, B, per_batch, 0)

    @pl.when(wid == NW - 1)
    def _():
        pltpu.sync_copy(posm_hbm.at[pl.ds(T, 2), :], pos_buf.at[pl.ds(0, 2)])

        def wr(b, _):
            lb = get_len(b)
            pltpu.sync_copy(x_hbm.at[b, pl.ds(T - 8, 8), :], x_buf.at[pl.ds(0, 8)])
            process_row(0, T, lb, 7)

            @pl.when(lb == T)
            def _():
                _fill_row(out_buf, 1,
                          lambda sl: eos_n[0, sl] * _f(PRE_SCALE) + pos_buf[1, sl])

            @pl.when(lb < T)
            def _():
                _fill_row(out_buf, 1, lambda sl: pad_row[0, sl] * _f(1.0))

            _row_normalize(out_buf, 1, gv, bv)
            pltpu.sync_copy(out_buf.at[pl.ds(0, 2)], out_hbm.at[b, pl.ds(T, 2), :])
            return 0

        lax.fori_loop(0, B, wr, 0)


def kernel(x, padding_mask, bos_emb, eos_emb, pos_table,
           ln_special_g, ln_special_b, ln_g, ln_b):
    lengths0 = (T - jnp.sum(padding_mask.astype(jnp.int32), axis=1)).astype(jnp.int32)

    posm = lax.slice(pos_table, (2, 0), (2 + TOUT, C))
    pad = lax.slice(pos_table, (PADDING_IDX, 0), (PADDING_IDX + 1, C))

    mesh = plsc.VectorSubcoreMesh(core_axis_name="c", subcore_axis_name="s")
    row = lambda a: a.reshape(1, C)

    sc = pl.kernel(
        _sc_body,
        mesh=mesh,
        compiler_params=pltpu.CompilerParams(needs_layout_passes=False),
        out_type=jax.ShapeDtypeStruct((B, TOUT, C), jnp.float32),
        scratch_types=[
            pltpu.VMEM((R + 8, C), jnp.float32),
            pltpu.VMEM((R, C), jnp.float32),
            pltpu.VMEM((R, C), jnp.float32),
            pltpu.VMEM((1, C), jnp.float32),
            pltpu.VMEM((1, C), jnp.float32),
            pltpu.VMEM((1, C), jnp.float32),
            pltpu.VMEM((1, C), jnp.float32),
            pltpu.VMEM((1, C), jnp.float32),
            pltpu.VMEM((1, C), jnp.float32),
            pltpu.VMEM((1, C), jnp.float32),
            pltpu.VMEM((B,), jnp.int32),
        ],
    )

    out = sc(x, lengths0, posm, pad, row(bos_emb), row(eos_emb),
             row(ln_special_g), row(ln_special_b), row(ln_g), row(ln_b))

    lengths = lengths0 + 2
    new_padding_mask = jnp.arange(TOUT, dtype=jnp.int32)[None, :] >= lengths[:, None]
    return (out, new_padding_mask, lengths)


# speedup vs baseline: 2.2524x; 2.2524x over previous
import jax
import jax.numpy as jnp
from jax import lax
from jax.experimental import pallas as pl
from jax.experimental.pallas import tpu as pltpu
from jax.experimental.pallas import tpu_sc as plsc

B, T, C = 16, 2048, 1024
TOUT = T + 2
PADDING_IDX = 1
PRE_SCALE = 7.0
EPS = 1e-5
L = 16
NJ = C // L
NW = 32
RPW = T // NW
CH = 8
NK = RPW // CH


def _f(x):
    return jnp.float32(x)


def _rsqrt_vec(av):
    ii = lax.bitcast_convert_type(av, jnp.int32)
    y = lax.bitcast_convert_type(jnp.int32(0x5F3759DF) - (ii >> 1), jnp.float32)
    for _ in range(3):
        y = y * (_f(1.5) - _f(0.5) * av * y * y)
    return y


def _row_normalize(buf, r, g_buf, b_buf):
    def stats(j, carry):
        s, ss = carry
        v = buf[r, pl.ds(j * L, L)]
        return s + v, ss + v * v
    z = jnp.zeros((L,), jnp.float32)
    s, ss = lax.fori_loop(0, NJ, stats, (z, z))
    mean = jnp.sum(s) * _f(1.0 / C)
    var = jnp.sum(ss) * _f(1.0 / C) - mean * mean
    inv = _rsqrt_vec(jnp.full((L,), var + _f(EPS), jnp.float32))
    mv = jnp.full((L,), mean, jnp.float32)

    def norm(j, _):
        sl = pl.ds(j * L, L)
        buf[r, sl] = (buf[r, sl] - mv) * inv * g_buf[0, sl] + b_buf[0, sl]
        return 0
    lax.fori_loop(0, NJ, norm, 0)


def _fill_row(dst, rd, fn):
    def body(j, _):
        sl = pl.ds(j * L, L)
        dst[rd, sl] = fn(sl)
        return 0
    lax.fori_loop(0, NJ, body, 0)


def _sc_body(x_hbm, len_hbm, posm_hbm, pad_hbm, bos_hbm, eos_hbm,
             gsp_hbm, bsp_hbm, g_hbm, b_hbm, out_hbm,
             lead_buf, xr0, xr1, or0, or1, pos_buf,
             gv, bv, gsv, bsv, bos_n, eos_n, pad_row, lenv,
             sem_l, sx0, sx1, so0, so1):
    wid = lax.axis_index("s") * 2 + lax.axis_index("c")
    tau = wid * RPW

    pltpu.sync_copy(g_hbm, gv)
    pltpu.sync_copy(b_hbm, bv)
    pltpu.sync_copy(gsp_hbm, gsv)
    pltpu.sync_copy(bsp_hbm, bsv)
    pltpu.sync_copy(bos_hbm, bos_n)
    pltpu.sync_copy(eos_hbm, eos_n)
    pltpu.sync_copy(pad_hbm, pad_row)
    pltpu.sync_copy(len_hbm, lenv)

    _row_normalize(bos_n, 0, gsv, bsv)
    _row_normalize(eos_n, 0, gsv, bsv)

    pltpu.sync_copy(posm_hbm.at[pl.ds(tau, RPW), :], pos_buf)

    def get_len(b):
        lbv = plsc.load_gather(lenv, [jnp.full((L,), b, jnp.int32)])
        return jnp.max(lbv)

    xrs = (xr0, xr1)
    ors = (or0, or1)
    xsems = (sx0, sx1)
    osems = (so0, so1)

    def per_batch(b, _):
        lb = get_len(b)

        lead_cp = pltpu.async_copy(
            x_hbm.at[b, pl.ds(pl.multiple_of(jnp.maximum(tau - 8, 0), 8), CH), :],
            lead_buf, sem_l)
        x_cps = [pltpu.async_copy(
            x_hbm.at[b, pl.ds(tau, CH), :], xr0.at[pl.ds(1, CH)], sx0)]
        out_cps = []

        for k in range(NK):
            d = k % 2
            xr = xrs[d]
            orr = ors[d]
            base = tau + k * CH

            x_cps[k].wait()
            if k == 0:
                lead_cp.wait()
                prev_src, prev_row_i = lead_buf, CH - 1
            else:
                prev_src, prev_row_i = xrs[1 - d], CH
            def cpy(j, _):
                sl = pl.ds(j * L, L)
                xr[0, sl] = prev_src[prev_row_i, sl]
                return 0
            lax.fori_loop(0, NJ, cpy, 0)

            if k + 1 < NK:
                x_cps.append(pltpu.async_copy(
                    x_hbm.at[b, pl.ds(tau + (k + 1) * CH, CH), :],
                    xrs[1 - d].at[pl.ds(1, CH)], xsems[1 - d]))
            if k >= 2:
                out_cps[k - 2].wait()

            def mat(j, _):
                sl = pl.ds(j * L, L)
                for r in range(CH):
                    orr[r, sl] = xr[r, sl] * _f(PRE_SCALE) + pos_buf[k * CH + r, sl]
                return 0
            lax.fori_loop(0, NJ, mat, 0)

            def fix(r, _):
                t = base + r

                @pl.when(t == 0)
                def _():
                    _fill_row(orr, r, lambda sl: bos_n[0, sl] * _f(PRE_SCALE)
                              + pos_buf[k * CH + r, sl])

                @pl.when(t == lb + 1)
                def _():
                    _fill_row(orr, r, lambda sl: eos_n[0, sl] * _f(PRE_SCALE)
                              + pos_buf[k * CH + r, sl])

                @pl.when(t > lb + 1)
                def _():
                    _fill_row(orr, r, lambda sl: xr[r, sl] * _f(PRE_SCALE)
                              + pad_row[0, sl])
                return 0
            lax.fori_loop(0, CH, fix, 0)

            def stats(j, carry):
                sl = pl.ds(j * L, L)
                out = []
                for r in range(CH):
                    v = orr[r, sl]
                    out.append(carry[2 * r] + v)
                    out.append(carry[2 * r + 1] + v * v)
                return tuple(out)
            z = jnp.zeros((L,), jnp.float32)
            acc = lax.fori_loop(0, NJ, stats, (z,) * (2 * CH))

            mv, iv = [], []
            for r in range(CH):
                mean = jnp.sum(acc[2 * r]) * _f(1.0 / C)
                var = jnp.sum(acc[2 * r + 1]) * _f(1.0 / C) - mean * mean
                mv.append(jnp.full((L,), mean, jnp.float32))
                iv.append(_rsqrt_vec(jnp.full((L,), var + _f(EPS), jnp.float32)))

            def norm(j, _):
                sl = pl.ds(j * L, L)
                gvec = gv[0, sl]
                bvec = bv[0, sl]
                for r in range(CH):
                    orr[r, sl] = (orr[r, sl] - mv[r]) * iv[r] * gvec + bvec
                return 0
            lax.fori_loop(0, NJ, norm, 0)

            out_cps.append(pltpu.async_copy(
                orr, out_hbm.at[b, pl.ds(base, CH), :], osems[d]))

        out_cps[NK - 2].wait()
        out_cps[NK - 1].wait()
        return 0

    lax.fori_loop(0, B, per_batch, 0)

    @pl.when(wid == NW - 1)
    def _():
        pltpu.sync_copy(posm_hbm.at[pl.ds(T, 2), :], pos_buf.at[pl.ds(0, 2)])

        def wr(b, _):
            lb = get_len(b)
            pltpu.sync_copy(x_hbm.at[b, pl.ds(T - CH, CH), :], lead_buf)

            _fill_row(or0, 0, lambda sl: lead_buf[CH - 1, sl] * _f(PRE_SCALE)
                      + pos_buf[0, sl])

            @pl.when(T == lb + 1)
            def _():
                _fill_row(or0, 0, lambda sl: eos_n[0, sl] * _f(PRE_SCALE)
                          + pos_buf[0, sl])

            @pl.when(T > lb + 1)
            def _():
                _fill_row(or0, 0, lambda sl: lead_buf[CH - 1, sl] * _f(PRE_SCALE)
                          + pad_row[0, sl])

            _row_normalize(or0, 0, gv, bv)

            @pl.when(lb == T)
            def _():
                _fill_row(or0, 1, lambda sl: eos_n[0, sl] * _f(PRE_SCALE)
                          + pos_buf[1, sl])

            @pl.when(lb < T)
            def _():
                _fill_row(or0, 1, lambda sl: pad_row[0, sl] * _f(1.0))

            _row_normalize(or0, 1, gv, bv)
            pltpu.sync_copy(or0.at[pl.ds(0, 2)], out_hbm.at[b, pl.ds(T, 2), :])
            return 0

        lax.fori_loop(0, B, wr, 0)


def kernel(x, padding_mask, bos_emb, eos_emb, pos_table,
           ln_special_g, ln_special_b, ln_g, ln_b):
    lengths0 = (T - jnp.sum(padding_mask.astype(jnp.int32), axis=1)).astype(jnp.int32)

    posm = lax.slice(pos_table, (2, 0), (2 + TOUT, C))
    pad = lax.slice(pos_table, (PADDING_IDX, 0), (PADDING_IDX + 1, C))

    mesh = plsc.VectorSubcoreMesh(core_axis_name="c", subcore_axis_name="s")
    row = lambda a: a.reshape(1, C)

    sc = pl.kernel(
        _sc_body,
        mesh=mesh,
        compiler_params=pltpu.CompilerParams(needs_layout_passes=False, use_tc_tiling_on_sc=False),
        out_type=jax.ShapeDtypeStruct((B, TOUT, C), jnp.float32),
        scratch_types=[
            pltpu.VMEM((CH, C), jnp.float32),
            pltpu.VMEM((CH + 1, C), jnp.float32),
            pltpu.VMEM((CH + 1, C), jnp.float32),
            pltpu.VMEM((CH, C), jnp.float32),
            pltpu.VMEM((CH, C), jnp.float32),
            pltpu.VMEM((RPW, C), jnp.float32),
            pltpu.VMEM((1, C), jnp.float32),
            pltpu.VMEM((1, C), jnp.float32),
            pltpu.VMEM((1, C), jnp.float32),
            pltpu.VMEM((1, C), jnp.float32),
            pltpu.VMEM((1, C), jnp.float32),
            pltpu.VMEM((1, C), jnp.float32),
            pltpu.VMEM((1, C), jnp.float32),
            pltpu.VMEM((B,), jnp.int32),
            pltpu.SemaphoreType.DMA,
            pltpu.SemaphoreType.DMA,
            pltpu.SemaphoreType.DMA,
            pltpu.SemaphoreType.DMA,
            pltpu.SemaphoreType.DMA,
        ],
    )

    out = sc(x, lengths0, posm, pad, row(bos_emb), row(eos_emb),
             row(ln_special_g), row(ln_special_b), row(ln_g), row(ln_b))

    lengths = lengths0 + 2
    new_padding_mask = jnp.arange(TOUT, dtype=jnp.int32)[None, :] >= lengths[:, None]
    return (out, new_padding_mask, lengths)
